# baseline (speedup 1.0000x reference)
import functools

import jax
import jax.numpy as jnp
from jax import lax
from jax.experimental import pallas as pl
from jax.experimental.pallas import tpu as pltpu
from jax.experimental.pallas import tpu_sc as plsc

_NUM_CLASSES = 56
_N_CTX = 4
_N_META = 16
_SUFFIX_LEN = 60
_DIM = 512
_BATCH = 4096
_ROWS = 1 + _N_CTX + _N_META + _SUFFIX_LEN

_NC = 2
_NS = 16
_NW = _NC * _NS
_BPW = _BATCH // _NW
_LANES = 16


def _body(lbl_hbm, cls_hbm, pre_hbm, meta_hbm, suf_hbm, out_hbm,
          lbl_v, idx_v, buf0, buf1, gs0, gs1, ss0, ss1):
    wid = lax.axis_index("s") * _NC + lax.axis_index("c")
    base = wid * _BPW

    pltpu.sync_copy(lbl_hbm.at[pl.ds(base, _BPW)], lbl_v)

    for g in range(_BPW // _LANES):
        l16 = lbl_v[pl.ds(g * _LANES, _LANES)]
        pos = jnp.arange(_LANES, dtype=jnp.int32) + (g * _LANES)
        for j in range(_N_CTX):
            plsc.store_scatter(
                idx_v,
                [pos, jnp.full((_LANES,), j, jnp.int32)],
                l16 * _N_CTX + j,
            )

    for buf in (buf0, buf1):
        pltpu.sync_copy(pre_hbm, buf.at[pl.ds(0, 1)])
        pltpu.sync_copy(meta_hbm, buf.at[pl.ds(1 + _N_CTX, _N_META)])
        pltpu.sync_copy(suf_hbm, buf.at[pl.ds(1 + _N_CTX + _N_META, _SUFFIX_LEN)])

    bufs = (buf0, buf1)
    gsems = (gs0, gs1)
    ssems = (ss0, ss1)

    for b in range(2):
        pltpu.async_copy(cls_hbm.at[idx_v.at[b]], bufs[b].at[pl.ds(1, _N_CTX)],
                         gsems[b])

    def loop_body(g, carry):
        for b in range(2):
            e = g * 2 + b
            pltpu.make_async_copy(cls_hbm.at[idx_v.at[e]],
                                  bufs[b].at[pl.ds(1, _N_CTX)], gsems[b]).wait()
            pltpu.async_copy(bufs[b], out_hbm.at[base + e], ssems[b])
        for b in range(2):
            e = g * 2 + b

            @pl.when(e + 2 < _BPW)
            def _refill():
                pltpu.make_async_copy(bufs[b], out_hbm.at[base + e],
                                      ssems[b]).wait()
                pltpu.async_copy(cls_hbm.at[idx_v.at[e + 2]],
                                 bufs[b].at[pl.ds(1, _N_CTX)], gsems[b])
        return carry

    lax.fori_loop(0, _BPW // 2, loop_body, None)

    for b in range(2):
        pltpu.make_async_copy(bufs[b], out_hbm.at[base], ssems[b]).wait()


_sc_call = functools.partial(
    pl.kernel,
    out_type=jax.ShapeDtypeStruct((_BATCH, _ROWS, _DIM), jnp.float32),
    mesh=plsc.VectorSubcoreMesh(core_axis_name="c", subcore_axis_name="s"),
    compiler_params=pltpu.CompilerParams(
        use_tc_tiling_on_sc=False, needs_layout_passes=False),
    scratch_types=[
        pltpu.VMEM((_BPW,), jnp.int32),
        pltpu.VMEM((_BPW, _N_CTX), jnp.int32),
        pltpu.VMEM((_ROWS, _DIM), jnp.float32),
        pltpu.VMEM((_ROWS, _DIM), jnp.float32),
        pltpu.SemaphoreType.DMA,
        pltpu.SemaphoreType.DMA,
        pltpu.SemaphoreType.DMA,
        pltpu.SemaphoreType.DMA,
    ],
)(_body)


def kernel(label, cls_ctx, meta_ctx, token_prefix, token_suffix):
    lbl = label.astype(jnp.int32)
    cls2 = cls_ctx.reshape(_NUM_CLASSES * _N_CTX, _DIM)
    pre2 = token_prefix.reshape(1, _DIM)
    meta2 = meta_ctx.reshape(_N_META, _DIM)
    suf2 = token_suffix.reshape(_SUFFIX_LEN, _DIM)
    return _sc_call(lbl, cls2, pre2, meta2, suf2)


# speedup vs baseline: 1.0794x; 1.0794x over previous
import functools

import jax
import jax.numpy as jnp
from jax import lax
from jax.experimental import pallas as pl
from jax.experimental.pallas import tpu as pltpu
from jax.experimental.pallas import tpu_sc as plsc

_NUM_CLASSES = 56
_N_CTX = 4
_N_META = 16
_SUFFIX_LEN = 60
_DIM = 512
_BATCH = 4096
_ROWS = 1 + _N_CTX + _N_META + _SUFFIX_LEN
_HEAD = 8
_TAIL = _ROWS - _HEAD

_NC = 2
_NS = 16
_NW = _NC * _NS
_BPW = _BATCH // _NW
_LANES = 16
_META_BASE = 1 + _NUM_CLASSES * _N_CTX


def _body(lbl_hbm, aug_hbm, tmpl_hbm, out_hbm,
          lbl_v, idx_v, tbuf, hbufa, hbufb, gsa, gsb, hsem, tsem):
    wid = lax.axis_index("s") * _NC + lax.axis_index("c")
    base = wid * _BPW

    pltpu.sync_copy(lbl_hbm.at[pl.ds(base, _BPW)], lbl_v)
    pltpu.sync_copy(tmpl_hbm, tbuf)

    lanes = jax.lax.iota(jnp.int32, _LANES)
    for g in range(_BPW // _LANES):
        lv = lbl_v[pl.ds(g * _LANES, _LANES)]
        pos_base = lanes * _HEAD + (g * _LANES * _HEAD)
        for r in range(_HEAD):
            if r == 0:
                vals = jnp.zeros((_LANES,), jnp.int32)
            elif r <= _N_CTX:
                vals = lv * _N_CTX + r
            else:
                vals = jnp.full((_LANES,), _META_BASE + (r - 5), jnp.int32)
            plsc.store_scatter(idx_v, [pos_base + r], vals)

    def heads(buf, e0):
        h0 = pltpu.async_copy(buf.at[pl.ds(0, _HEAD)],
                              out_hbm.at[base + e0, pl.ds(0, _HEAD)], hsem)
        h1 = pltpu.async_copy(buf.at[pl.ds(_HEAD, _HEAD)],
                              out_hbm.at[base + e0 + 1, pl.ds(0, _HEAD)], hsem)
        return h0, h1

    def loop_body(q, carry):
        e = q * 4
        @pl.when(q > 0)
        def _drain_tails():
            for _ in range(4):
                pltpu.make_async_copy(
                    tbuf, out_hbm.at[base, pl.ds(_HEAD, _TAIL)], tsem).wait()

        off_a = pl.multiple_of(e * _HEAD, _LANES)
        off_b = pl.multiple_of((e + 2) * _HEAD, _LANES)
        ga = pltpu.async_copy(aug_hbm.at[idx_v.at[pl.ds(off_a, _LANES)]],
                              hbufa, gsa)
        gb = pltpu.async_copy(aug_hbm.at[idx_v.at[pl.ds(off_b, _LANES)]],
                              hbufb, gsb)
        for k in range(4):
            pltpu.async_copy(tbuf,
                             out_hbm.at[base + e + k, pl.ds(_HEAD, _TAIL)],
                             tsem)
        ga.wait()
        ha0, ha1 = heads(hbufa, e)
        gb.wait()
        hb0, hb1 = heads(hbufb, e + 2)
        ha0.wait()
        ha1.wait()
        hb0.wait()
        hb1.wait()
        return carry

    lax.fori_loop(0, _BPW // 4, loop_body, None)

    for _ in range(4):
        pltpu.make_async_copy(
            tbuf, out_hbm.at[base, pl.ds(_HEAD, _TAIL)], tsem).wait()


_sc_call = functools.partial(
    pl.kernel,
    out_type=jax.ShapeDtypeStruct((_BATCH, _ROWS, _DIM), jnp.float32),
    mesh=plsc.VectorSubcoreMesh(core_axis_name="c", subcore_axis_name="s"),
    compiler_params=pltpu.CompilerParams(needs_layout_passes=False),
    scratch_types=[
        pltpu.VMEM((_BPW,), jnp.int32),
        pltpu.VMEM((_BPW * _HEAD,), jnp.int32),
        pltpu.VMEM((_TAIL, _DIM), jnp.float32),
        pltpu.VMEM((2 * _HEAD, _DIM), jnp.float32),
        pltpu.VMEM((2 * _HEAD, _DIM), jnp.float32),
        pltpu.SemaphoreType.DMA,
        pltpu.SemaphoreType.DMA,
        pltpu.SemaphoreType.DMA,
        pltpu.SemaphoreType.DMA,
    ],
)(_body)


def kernel(label, cls_ctx, meta_ctx, token_prefix, token_suffix):
    lbl = label.astype(jnp.int32)
    cls2 = cls_ctx.reshape(_NUM_CLASSES * _N_CTX, _DIM)
    pre2 = token_prefix.reshape(1, _DIM)
    meta2 = meta_ctx.reshape(_N_META, _DIM)
    suf2 = token_suffix.reshape(_SUFFIX_LEN, _DIM)
    aug = jnp.concatenate([pre2, cls2, meta2], axis=0)
    tmpl = jnp.concatenate([meta2[3:], suf2], axis=0)
    return _sc_call(lbl, aug, tmpl)


# speedup vs baseline: 1.1030x; 1.0219x over previous
import functools

import jax
import jax.numpy as jnp
from jax import lax
from jax.experimental import pallas as pl
from jax.experimental.pallas import tpu as pltpu
from jax.experimental.pallas import tpu_sc as plsc

_NUM_CLASSES = 56
_N_CTX = 4
_N_META = 16
_SUFFIX_LEN = 60
_DIM = 512
_BATCH = 4096
_ROWS = 1 + _N_CTX + _N_META + _SUFFIX_LEN
_HEAD = 8
_BLK = _ROWS - 1

_NC = 2
_NS = 16
_NW = _NC * _NS
_BPW = _BATCH // _NW
_LANES = 16
_META_BASE = 1 + _NUM_CLASSES * _N_CTX


def _body(lbl_hbm, aug_hbm, tmpl_hbm, last_hbm, out_hbm,
          lbl_v, idx_v, bbuf0, bbuf1, sbuf,
          gs0, gs1, ws0, ws1, rs):
    wid = lax.axis_index("s") * _NC + lax.axis_index("c")
    base = wid * _BPW

    pltpu.sync_copy(lbl_hbm.at[pl.ds(base, _BPW)], lbl_v)
    pltpu.sync_copy(tmpl_hbm, bbuf0.at[pl.ds(_HEAD, _BLK - _HEAD)])
    pltpu.sync_copy(tmpl_hbm, bbuf1.at[pl.ds(_HEAD, _BLK - _HEAD)])
    pltpu.sync_copy(last_hbm, sbuf)

    lanes = jax.lax.iota(jnp.int32, _LANES)
    for g in range(_BPW // _LANES):
        lv = lbl_v[pl.ds(g * _LANES, _LANES)]
        pos_base = lanes * _HEAD + (g * _LANES * _HEAD)
        for r in range(_HEAD):
            if r == 0:
                vals = jnp.zeros((_LANES,), jnp.int32)
            elif r <= _N_CTX:
                vals = lv * _N_CTX + r
            else:
                vals = jnp.full((_LANES,), _META_BASE + (r - 5), jnp.int32)
            plsc.store_scatter(idx_v, [pos_base + r], vals)

    bufs = (bbuf0, bbuf1)
    gsems = (gs0, gs1)
    wsems = (ws0, ws1)

    def gather(e, buf, sem):
        off = pl.multiple_of(e * _HEAD, _HEAD)
        return pltpu.async_copy(aug_hbm.at[idx_v.at[pl.ds(off, _HEAD)]],
                                buf.at[pl.ds(0, _HEAD)], sem)

    for s in range(2):
        gather(s, bufs[s], gsems[s])

    def loop_body(g, carry):
        for s in range(2):
            e = g * 2 + s
            pltpu.make_async_copy(aug_hbm.at[pl.ds(0, _HEAD)],
                                  bufs[s].at[pl.ds(0, _HEAD)], gsems[s]).wait()
            pltpu.async_copy(bufs[s], out_hbm.at[base + e, pl.ds(0, _BLK)],
                             wsems[s])
            pltpu.async_copy(sbuf, out_hbm.at[base + e, pl.ds(_BLK, 1)], rs)
        @pl.when(g > 0)
        def _drain_last_row():
            for _ in range(2):
                pltpu.make_async_copy(
                    sbuf, out_hbm.at[base, pl.ds(_BLK, 1)], rs).wait()
        for s in range(2):
            e = g * 2 + s

            @pl.when(e + 2 < _BPW)
            def _refill():
                pltpu.make_async_copy(
                    bufs[s], out_hbm.at[base, pl.ds(0, _BLK)], wsems[s]).wait()
                gather(e + 2, bufs[s], gsems[s])
        return carry

    lax.fori_loop(0, _BPW // 2, loop_body, None)

    for s in range(2):
        pltpu.make_async_copy(
            bufs[s], out_hbm.at[base, pl.ds(0, _BLK)], wsems[s]).wait()
        pltpu.make_async_copy(
            sbuf, out_hbm.at[base, pl.ds(_BLK, 1)], rs).wait()


_sc_call = functools.partial(
    pl.kernel,
    out_type=jax.ShapeDtypeStruct((_BATCH, _ROWS, _DIM), jnp.float32),
    mesh=plsc.VectorSubcoreMesh(core_axis_name="c", subcore_axis_name="s"),
    compiler_params=pltpu.CompilerParams(needs_layout_passes=False),
    scratch_types=[
        pltpu.VMEM((_BPW,), jnp.int32),
        pltpu.VMEM((_BPW * _HEAD,), jnp.int32),
        pltpu.VMEM((_BLK, _DIM), jnp.float32),
        pltpu.VMEM((_BLK, _DIM), jnp.float32),
        pltpu.VMEM((1, _DIM), jnp.float32),
        pltpu.SemaphoreType.DMA,
        pltpu.SemaphoreType.DMA,
        pltpu.SemaphoreType.DMA,
        pltpu.SemaphoreType.DMA,
        pltpu.SemaphoreType.DMA,
    ],
)(_body)


def kernel(label, cls_ctx, meta_ctx, token_prefix, token_suffix):
    lbl = label.astype(jnp.int32)
    cls2 = cls_ctx.reshape(_NUM_CLASSES * _N_CTX, _DIM)
    pre2 = token_prefix.reshape(1, _DIM)
    meta2 = meta_ctx.reshape(_N_META, _DIM)
    suf2 = token_suffix.reshape(_SUFFIX_LEN, _DIM)
    aug = jnp.concatenate([pre2, cls2, meta2], axis=0)
    tmpl = jnp.concatenate([meta2[3:], suf2[:-1]], axis=0)
    last = suf2[-1:]
    return _sc_call(lbl, aug, tmpl, last)


# speedup vs baseline: 4.3364x; 3.9314x over previous
import functools

import jax
import jax.numpy as jnp
from jax import lax
from jax.experimental import pallas as pl
from jax.experimental.pallas import tpu as pltpu
from jax.experimental.pallas import tpu_sc as plsc

_NUM_CLASSES = 56
_N_CTX = 4
_N_META = 16
_SUFFIX_LEN = 60
_DIM = 512
_BATCH = 4096
_ROWS = 1 + _N_CTX + _N_META + _SUFFIX_LEN
_NBC = _ROWS - _N_CTX

_NC = 2
_NS = 16
_NW = _NC * _NS
_BPW = _BATCH // _NW
_LANES = 16

_CCH = 32
_CUN = _N_CTX * (_BPW // _CCH)
_REP = 64
_QTR = 1024
_BUN = _NBC * (_BATCH // _QTR)
_BPWK = 10


def _body(lbl_hbm, cls_hbm, rep_hbm, out_hbm,
          lbl_v, idx_v, cbuf0, cbuf1, bbuf0, bbuf1,
          gs0, gs1, cs0, cs1, ws0, ws1):
    wid = lax.axis_index("s") * _NC + lax.axis_index("c")
    base = wid * _BPW

    pltpu.sync_copy(lbl_hbm.at[pl.ds(base, _BPW)], lbl_v)

    lanes = jax.lax.iota(jnp.int32, _LANES)
    for g in range(_BPW // _LANES):
        lv = lbl_v[pl.ds(g * _LANES, _LANES)]
        for rr in range(_N_CTX):
            plsc.store_scatter(idx_v, [lanes + (rr * _BPW + g * _LANES)],
                               lv * _N_CTX + rr)

    cbufs = (cbuf0, cbuf1)
    gsems = (gs0, gs1)
    csems = (cs0, cs1)

    def cls_gather(i, buf, sem):
        off = pl.multiple_of(i * _CCH, _CCH)
        return pltpu.async_copy(cls_hbm.at[idx_v.at[pl.ds(off, _CCH)]], buf,
                                sem)

    def cls_dst(i):
        row = 1 + i // (_BPW // _CCH)
        boff = base + _CCH * (i % (_BPW // _CCH))
        return out_hbm.at[row, pl.ds(boff, _CCH)]

    for s in range(2):
        cls_gather(s, cbufs[s], gsems[s])
    for i in range(_CUN):
        s = i % 2
        pltpu.make_async_copy(cls_hbm.at[pl.ds(0, _CCH)], cbufs[s],
                              gsems[s]).wait()
        pltpu.async_copy(cbufs[s], cls_dst(i), csems[s])
        if i + 2 < _CUN:
            pltpu.make_async_copy(cbufs[s], cls_dst(i), csems[s]).wait()
            cls_gather(i + 2, cbufs[s], gsems[s])

    bbufs = (bbuf0, bbuf1)
    wsems = (ws0, ws1)
    for t in range(_BPWK):
        s = t % 2
        u = wid + _NW * t
        u = jnp.where(u < _BUN, u, u - _BUN)
        j = u // (_BATCH // _QTR)
        qtr = u % (_BATCH // _QTR)
        row = jnp.where(j == 0, 0, j + _N_CTX)
        if t >= 2:
            for _ in range(_QTR // _REP):
                pltpu.make_async_copy(
                    bbufs[s], out_hbm.at[0, pl.ds(0, _REP)], wsems[s]).wait()
        pltpu.sync_copy(rep_hbm.at[j], bbufs[s])
        for k in range(_QTR // _REP):
            pltpu.async_copy(bbufs[s],
                             out_hbm.at[row, pl.ds(qtr * _QTR + k * _REP,
                                                   _REP)],
                             wsems[s])

    for s in range(2):
        pltpu.make_async_copy(cbufs[s], cls_dst(0), csems[s]).wait()
        for _ in range(_QTR // _REP):
            pltpu.make_async_copy(
                bbufs[s], out_hbm.at[0, pl.ds(0, _REP)], wsems[s]).wait()


_sc_call = functools.partial(
    pl.kernel,
    out_type=jax.ShapeDtypeStruct((_ROWS, _BATCH, _DIM), jnp.float32),
    mesh=plsc.VectorSubcoreMesh(core_axis_name="c", subcore_axis_name="s"),
    compiler_params=pltpu.CompilerParams(needs_layout_passes=False),
    scratch_types=[
        pltpu.VMEM((_BPW,), jnp.int32),
        pltpu.VMEM((_N_CTX * _BPW,), jnp.int32),
        pltpu.VMEM((_CCH, _DIM), jnp.float32),
        pltpu.VMEM((_CCH, _DIM), jnp.float32),
        pltpu.VMEM((_REP, _DIM), jnp.float32),
        pltpu.VMEM((_REP, _DIM), jnp.float32),
        pltpu.SemaphoreType.DMA,
        pltpu.SemaphoreType.DMA,
        pltpu.SemaphoreType.DMA,
        pltpu.SemaphoreType.DMA,
        pltpu.SemaphoreType.DMA,
        pltpu.SemaphoreType.DMA,
    ],
)(_body)


def kernel(label, cls_ctx, meta_ctx, token_prefix, token_suffix):
    lbl = label.astype(jnp.int32)
    cls2 = cls_ctx.reshape(_NUM_CLASSES * _N_CTX, _DIM)
    pre2 = token_prefix.reshape(1, _DIM)
    meta2 = meta_ctx.reshape(_N_META, _DIM)
    suf2 = token_suffix.reshape(_SUFFIX_LEN, _DIM)
    brows = jnp.concatenate([pre2, meta2, suf2], axis=0)
    rep = jnp.broadcast_to(brows[:, None, :], (_NBC, _REP, _DIM))
    rep = jnp.reshape(rep, (_NBC, _REP, _DIM))
    out = _sc_call(lbl, cls2, rep)
    return jnp.transpose(out, (1, 0, 2))
